# baseline (speedup 1.0000x reference)
import functools

import jax
import jax.numpy as jnp
from jax import lax
from jax.experimental import pallas as pl
from jax.experimental.pallas import tpu as pltpu
from jax.experimental.pallas import tpu_sc as plsc

BATCH = 4096
HIST = 200
VOCAB = 100000
DIM = 16
OUT = 8
N = BATCH * HIST

NUM_CORES = 2
NUM_SUBCORES = 16
NUM_WORKERS = NUM_CORES * NUM_SUBCORES
BAGS_PER_W = BATCH // NUM_WORKERS
IDX_PER_W = BAGS_PER_W * HIST
OUT_PER_W = BAGS_PER_W * OUT

CHUNK_A = 128
CHUNK_B = HIST - CHUNK_A
NBUF = 4

PACK = 128 // DIM
PROJ_ROWS = VOCAB // PACK


def _tc_project(table2, big_w):
    def proj_kernel(t_ref, w_ref, o_ref):
        o_ref[...] = jnp.dot(t_ref[...], w_ref[...],
                             preferred_element_type=jnp.float32)

    return pl.pallas_call(
        proj_kernel,
        out_shape=jax.ShapeDtypeStruct((PROJ_ROWS, 128), jnp.float32),
    )(table2, big_w)


def _tc_stage_idx(indices):
    def copy_kernel(x_ref, o_ref):
        o_ref[...] = x_ref[...]

    out = pl.pallas_call(
        copy_kernel,
        out_shape=jax.ShapeDtypeStruct((N // 128, 128), jnp.int32),
    )(indices.reshape(N // 128, 128))
    return out.reshape(N)


def _sc_bag_kernel():
    mesh = plsc.VectorSubcoreMesh(core_axis_name="c", subcore_axis_name="s")

    @functools.partial(
        pl.kernel,
        mesh=mesh,
        out_type=jax.ShapeDtypeStruct((BATCH * OUT,), jnp.float32),
        compiler_params=pltpu.CompilerParams(use_tc_tiling_on_sc=False,
                                             needs_layout_passes=False),
        scratch_types=(
            [pltpu.VMEM((IDX_PER_W,), jnp.int32)]
            + [pltpu.VMEM((HIST, DIM), jnp.float32) for _ in range(NBUF)]
            + [pltpu.VMEM((16,), jnp.float32),
               pltpu.VMEM((OUT_PER_W,), jnp.float32)]
            + [pltpu.SemaphoreType.DMA for _ in range(NBUF)]
        ),
    )
    def sc_kernel(idx_hbm, proj_hbm, bias2_hbm, out_hbm, idx_v,
                  buf0, buf1, buf2, buf3, bias_v, out_v,
                  sem0, sem1, sem2, sem3):
        wid = lax.axis_index("s") * NUM_CORES + lax.axis_index("c")
        bufs = (buf0, buf1, buf2, buf3)
        sems = (sem0, sem1, sem2, sem3)

        idx_base = pl.multiple_of(wid * IDX_PER_W, 8)
        pltpu.sync_copy(idx_hbm.at[pl.ds(idx_base, IDX_PER_W)], idx_v)
        pltpu.sync_copy(bias2_hbm, bias_v)

        def fire(bag, buf, sem):
            off = pl.multiple_of(bag * HIST, 8)
            pltpu.async_copy(
                proj_hbm.at[idx_v.at[pl.ds(off, CHUNK_A)]],
                buf.at[pl.ds(0, CHUNK_A)], sem)
            pltpu.async_copy(
                proj_hbm.at[idx_v.at[pl.ds(off + CHUNK_A, CHUNK_B)]],
                buf.at[pl.ds(CHUNK_A, CHUNK_B)], sem)

        def drain(buf, sem):
            pltpu.make_async_copy(
                proj_hbm.at[idx_v.at[pl.ds(0, CHUNK_A)]],
                buf.at[pl.ds(0, CHUNK_A)], sem).wait()
            pltpu.make_async_copy(
                proj_hbm.at[idx_v.at[pl.ds(0, CHUNK_B)]],
                buf.at[pl.ds(CHUNK_A, CHUNK_B)], sem).wait()

        for b in range(NBUF - 1):
            fire(b, bufs[b], sems[b])

        left_mask = lax.iota(jnp.int32, 16) < 8
        bias_vec = bias_v[...]

        def bag_sum(buf):
            accs = [buf[u] for u in range(8)]
            for j in range(1, HIST // 8):
                base = j * 8
                accs = [accs[u] + buf[base + u] for u in range(8)]
            s01 = accs[0] + accs[1]
            s23 = accs[2] + accs[3]
            s45 = accs[4] + accs[5]
            s67 = accs[6] + accs[7]
            return (s01 + s23) + (s45 + s67)

        def quad_body(i, _):
            acc_even = None
            for p in range(NBUF):
                bag = i * NBUF + p
                nxt = (p + NBUF - 1) % NBUF

                @pl.when(bag + NBUF - 1 < BAGS_PER_W)
                def _():
                    fire(bag + NBUF - 1, bufs[nxt], sems[nxt])

                drain(bufs[p], sems[p])
                acc = bag_sum(bufs[p])
                if p % 2 == 0:
                    acc_even = acc
                else:
                    tot = jnp.where(left_mask, acc_even, acc) + bias_vec
                    pair = i * 2 + p // 2
                    out_v[pl.ds(pl.multiple_of(pair * 16, 8), 16)] = tot
            return ()

        lax.fori_loop(0, BAGS_PER_W // NBUF, quad_body, (), unroll=False)

        out_base = pl.multiple_of(wid * OUT_PER_W, 8)
        pltpu.sync_copy(out_v, out_hbm.at[pl.ds(out_base, OUT_PER_W)])

    return sc_kernel


def kernel(indices, offsets, emb_table, fc_w, fc_b):
    del offsets
    w16 = jnp.concatenate([fc_w.T, fc_w.T], axis=1) * jnp.float32(1.0 / HIST)
    big_w = jnp.kron(jnp.eye(PACK, dtype=jnp.float32), w16)
    table2 = emb_table.reshape(PROJ_ROWS, 128)
    proj16 = _tc_project(table2, big_w).reshape(VOCAB, DIM)
    bias2 = jnp.concatenate([fc_b, fc_b]).astype(jnp.float32)
    sc = _sc_bag_kernel()
    return sc(_tc_stage_idx(indices), proj16, bias2).reshape(BATCH, OUT)
